# Initial kernel scaffold; baseline (speedup 1.0000x reference)
import jax
import jax.numpy as jnp
from jax.experimental import pallas as pl


def kernel(x, edge_index, W1, b1, W2, b2):
    raise NotImplementedError("write your pallas kernel here")


# speedup vs baseline: 19.6017x; 19.6017x over previous
import functools

import jax
import jax.numpy as jnp
from jax import lax
from jax.experimental import pallas as pl
from jax.experimental.pallas import tpu as pltpu
from jax.experimental.pallas import tpu_sc as plsc

N_NODES = 10000
N_EDGES = 320000
NC = 2
NS = 16
NW = NC * NS
CHUNK = 80
NCHUNK = N_EDGES // (NW * CHUNK)
ACC_ROWS = 10240
RPT = ACC_ROWS // NS
N_INIT = N_NODES // NS
BLK = 256
NBLK = ACC_ROWS // BLK

_sc_mesh = functools.partial(
    plsc.VectorSubcoreMesh, core_axis_name="c", subcore_axis_name="s"
)


@functools.partial(
    pl.kernel,
    out_type=jax.ShapeDtypeStruct((NC, ACC_ROWS), jnp.float32),
    mesh=_sc_mesh(),
    scratch_types=[
        pltpu.VMEM((NCHUNK, CHUNK), jnp.int32),
        pltpu.VMEM((CHUNK,), jnp.float32),
        pltpu.VMEM((RPT,), jnp.float32),
        pltpu.VMEM_SHARED((ACC_ROWS,), jnp.float32),
        pltpu.SemaphoreType.DMA,
    ],
)
def _deg_kernel(dst_hbm, out_hbm, dst_v, ones_v, zero_v, acc, sem):
    c = lax.axis_index("c")
    s = lax.axis_index("s")
    wid = s * NC + c

    pltpu.sync_copy(dst_hbm.at[wid], dst_v)
    for i in range(CHUNK // 16):
        ones_v[pl.ds(i * 16, 16)] = jnp.full((16,), 1.0, jnp.float32)

    def zfill(i, carry):
        zero_v[pl.ds(i * 16, 16)] = jnp.zeros((16,), jnp.float32)
        return carry

    lax.fori_loop(0, RPT // 16, zfill, 0)
    pltpu.sync_copy(zero_v, acc.at[pl.ds(s * RPT, RPT)])
    plsc.subcore_barrier()

    def step(j, carry):
        pltpu.sync_copy(ones_v, acc.at[dst_v.at[j]], add=True)
        return carry

    lax.fori_loop(0, NCHUNK, step, 0)
    plsc.subcore_barrier()
    pltpu.sync_copy(acc.at[pl.ds(s * RPT, RPT)], out_hbm.at[c, pl.ds(s * RPT, RPT)])


def _make_agg(d):
    @functools.partial(
        pl.kernel,
        out_type=jax.ShapeDtypeStruct((NC, ACC_ROWS, d), jnp.float32),
        mesh=_sc_mesh(),
        scratch_types=[
            pltpu.VMEM((NCHUNK, CHUNK), jnp.int32),
            pltpu.VMEM((NCHUNK, CHUNK), jnp.int32),
            pltpu.VMEM((CHUNK, d), jnp.float32),
            pltpu.VMEM_SHARED((ACC_ROWS, d), jnp.float32),
            pltpu.SemaphoreType.DMA,
        ],
        compiler_params=pltpu.CompilerParams(use_tc_tiling_on_sc=False),
    )
    def agg(h_hbm, src_hbm, dst_hbm, out_hbm, src_v, dst_v, rows_v, acc, sem):
        c = lax.axis_index("c")
        s = lax.axis_index("s")
        wid = s * NC + c

        pltpu.sync_copy(src_hbm.at[wid], src_v)
        pltpu.sync_copy(dst_hbm.at[wid], dst_v)

        def zrow(i, carry):
            for k in range(d // 16):
                rows_v[i, pl.ds(k * 16, 16)] = jnp.zeros((16,), jnp.float32)
            return carry

        lax.fori_loop(0, CHUNK, zrow, 0)

        @pl.when(c == 0)
        def _():
            @pl.when(s < NS - 1)
            def _():
                pltpu.sync_copy(
                    h_hbm.at[pl.ds(s * RPT, RPT)], acc.at[pl.ds(s * RPT, RPT)]
                )

            @pl.when(s == NS - 1)
            def _():
                last = N_NODES - (NS - 1) * RPT
                pltpu.sync_copy(
                    h_hbm.at[pl.ds((NS - 1) * RPT, last)],
                    acc.at[pl.ds((NS - 1) * RPT, last)],
                )

                def ztail(t, carry):
                    pltpu.sync_copy(
                        rows_v, acc.at[pl.ds(N_NODES + t * CHUNK, CHUNK)]
                    )
                    return carry

                lax.fori_loop(0, (ACC_ROWS - N_NODES) // CHUNK, ztail, 0)

        @pl.when(c == 1)
        def _():
            def zslice(t, carry):
                pltpu.sync_copy(rows_v, acc.at[pl.ds(s * RPT + t * CHUNK, CHUNK)])
                return carry

            lax.fori_loop(0, RPT // CHUNK, zslice, 0)

        plsc.subcore_barrier()

        def step(j, carry):
            pltpu.async_copy(h_hbm.at[src_v.at[j]], rows_v, sem).wait()
            pltpu.sync_copy(rows_v, acc.at[dst_v.at[j]], add=True)
            return carry

        lax.fori_loop(0, NCHUNK, step, 0)
        plsc.subcore_barrier()
        pltpu.sync_copy(
            acc.at[pl.ds(s * RPT, RPT)], out_hbm.at[c, pl.ds(s * RPT, RPT)]
        )

    return agg


_agg128 = _make_agg(128)
_agg64 = _make_agg(64)


def _mm1_body(x_ref, w_ref, dp_ref, o_ref):
    deg = dp_ref[0] + dp_ref[1] + 1.0
    dis = lax.rsqrt(deg)
    h = jnp.dot(x_ref[...], w_ref[...], preferred_element_type=jnp.float32)
    o_ref[...] = h * dis


def _mm1(x, W1, dpr):
    return pl.pallas_call(
        _mm1_body,
        grid=(NBLK,),
        in_specs=[
            pl.BlockSpec((BLK, 128), lambda i: (i, 0)),
            pl.BlockSpec((128, 128), lambda i: (0, 0)),
            pl.BlockSpec((NC, BLK, 1), lambda i: (0, i, 0)),
        ],
        out_specs=pl.BlockSpec((BLK, 128), lambda i: (i, 0)),
        out_shape=jax.ShapeDtypeStruct((ACC_ROWS, 128), jnp.float32),
    )(x, W1, dpr)


def _mm2_body(p_ref, dp_ref, b1_ref, w_ref, o_ref):
    deg = dp_ref[0] + dp_ref[1] + 1.0
    dis = lax.rsqrt(deg)
    aggd = p_ref[0] + p_ref[1]
    h = jnp.maximum(aggd * dis + b1_ref[...], 0.0)
    o_ref[...] = jnp.dot(h, w_ref[...], preferred_element_type=jnp.float32) * dis


def _mm2(p, dpr, b1, W2):
    return pl.pallas_call(
        _mm2_body,
        grid=(NBLK,),
        in_specs=[
            pl.BlockSpec((NC, BLK, 128), lambda i: (0, i, 0)),
            pl.BlockSpec((NC, BLK, 1), lambda i: (0, i, 0)),
            pl.BlockSpec((1, 128), lambda i: (0, 0)),
            pl.BlockSpec((128, 64), lambda i: (0, 0)),
        ],
        out_specs=pl.BlockSpec((BLK, 64), lambda i: (i, 0)),
        out_shape=jax.ShapeDtypeStruct((ACC_ROWS, 64), jnp.float32),
    )(p, dpr, b1, W2)


def _fin_body(q_ref, dp_ref, b2_ref, o_ref):
    deg = dp_ref[0] + dp_ref[1] + 1.0
    dis = lax.rsqrt(deg)
    o_ref[...] = (q_ref[0] + q_ref[1]) * dis + b2_ref[...]


def _fin(q, dpr, b2):
    return pl.pallas_call(
        _fin_body,
        grid=(NBLK,),
        in_specs=[
            pl.BlockSpec((NC, BLK, 64), lambda i: (0, i, 0)),
            pl.BlockSpec((NC, BLK, 1), lambda i: (0, i, 0)),
            pl.BlockSpec((1, 64), lambda i: (0, 0)),
        ],
        out_specs=pl.BlockSpec((BLK, 64), lambda i: (i, 0)),
        out_shape=jax.ShapeDtypeStruct((N_NODES, 64), jnp.float32),
    )(q, dpr, b2)


@jax.jit
def kernel(x, edge_index, W1, b1, W2, b2):
    src = edge_index[0].astype(jnp.int32).reshape(NW, NCHUNK, CHUNK)
    dst = edge_index[1].astype(jnp.int32).reshape(NW, NCHUNK, CHUNK)
    dp = _deg_kernel(dst)
    dpr = dp.reshape(NC, ACC_ROWS, 1)
    h1p = _mm1(x, W1, dpr)
    p = _agg128(h1p, src, dst)
    h2p = _mm2(p, dpr, b1.reshape(1, 128), W2)
    q = _agg64(h2p, src, dst)
    return _fin(q, dpr, b2.reshape(1, 64))


# speedup vs baseline: 27.9459x; 1.4257x over previous
import functools

import jax
import jax.numpy as jnp
from jax import lax
from jax.experimental import pallas as pl
from jax.experimental.pallas import tpu as pltpu
from jax.experimental.pallas import tpu_sc as plsc

N_NODES = 10000
N_EDGES = 320000
NC = 2
NS = 16
NW = NC * NS
CHUNK = 80
NCHUNK = N_EDGES // (NW * CHUNK)
ACC_ROWS = 10240
RPT = ACC_ROWS // NS
N_INIT = N_NODES // NS
BLK = 256
NBLK = ACC_ROWS // BLK

_sc_mesh = functools.partial(
    plsc.VectorSubcoreMesh, core_axis_name="c", subcore_axis_name="s"
)


@functools.partial(
    pl.kernel,
    out_type=jax.ShapeDtypeStruct((NC, ACC_ROWS), jnp.float32),
    mesh=_sc_mesh(),
    scratch_types=[
        pltpu.VMEM((NCHUNK, CHUNK), jnp.int32),
        pltpu.VMEM((CHUNK,), jnp.float32),
        pltpu.VMEM((RPT,), jnp.float32),
        pltpu.VMEM_SHARED((ACC_ROWS,), jnp.float32),
        pltpu.SemaphoreType.DMA,
    ],
)
def _deg_kernel(dst_hbm, out_hbm, dst_v, ones_v, zero_v, acc, sem):
    c = lax.axis_index("c")
    s = lax.axis_index("s")
    wid = s * NC + c

    pltpu.sync_copy(dst_hbm.at[wid], dst_v)
    for i in range(CHUNK // 16):
        ones_v[pl.ds(i * 16, 16)] = jnp.full((16,), 1.0, jnp.float32)

    def zfill(i, carry):
        zero_v[pl.ds(i * 16, 16)] = jnp.zeros((16,), jnp.float32)
        return carry

    lax.fori_loop(0, RPT // 16, zfill, 0)
    pltpu.sync_copy(zero_v, acc.at[pl.ds(s * RPT, RPT)])
    plsc.subcore_barrier()

    def step(j, carry):
        pltpu.sync_copy(ones_v, acc.at[dst_v.at[j]], add=True)
        return carry

    lax.fori_loop(0, NCHUNK, step, 0)
    plsc.subcore_barrier()
    pltpu.sync_copy(acc.at[pl.ds(s * RPT, RPT)], out_hbm.at[c, pl.ds(s * RPT, RPT)])


def _make_agg(d):
    @functools.partial(
        pl.kernel,
        out_type=jax.ShapeDtypeStruct((NC, ACC_ROWS, d), jnp.float32),
        mesh=_sc_mesh(),
        scratch_types=[
            pltpu.VMEM((NCHUNK, CHUNK), jnp.int32),
            pltpu.VMEM((NCHUNK, CHUNK), jnp.int32),
            pltpu.VMEM((CHUNK, d), jnp.float32),
            pltpu.VMEM((CHUNK, d), jnp.float32),
            pltpu.VMEM_SHARED((ACC_ROWS, d), jnp.float32),
            pltpu.SemaphoreType.DMA,
            pltpu.SemaphoreType.DMA,
        ],
        compiler_params=pltpu.CompilerParams(use_tc_tiling_on_sc=False),
    )
    def agg(h_hbm, src_hbm, dst_hbm, out_hbm, src_v, dst_v, rows_v, rows_w, acc, sem0, sem1):
        c = lax.axis_index("c")
        s = lax.axis_index("s")
        wid = s * NC + c

        pltpu.sync_copy(src_hbm.at[wid], src_v)
        pltpu.sync_copy(dst_hbm.at[wid], dst_v)

        def zrow(i, carry):
            for k in range(d // 16):
                rows_v[i, pl.ds(k * 16, 16)] = jnp.zeros((16,), jnp.float32)
            return carry

        lax.fori_loop(0, CHUNK, zrow, 0)

        @pl.when(c == 0)
        def _():
            @pl.when(s < NS - 1)
            def _():
                pltpu.sync_copy(
                    h_hbm.at[pl.ds(s * RPT, RPT)], acc.at[pl.ds(s * RPT, RPT)]
                )

            @pl.when(s == NS - 1)
            def _():
                last = N_NODES - (NS - 1) * RPT
                pltpu.sync_copy(
                    h_hbm.at[pl.ds((NS - 1) * RPT, last)],
                    acc.at[pl.ds((NS - 1) * RPT, last)],
                )

                def ztail(t, carry):
                    pltpu.sync_copy(
                        rows_v, acc.at[pl.ds(N_NODES + t * CHUNK, CHUNK)]
                    )
                    return carry

                lax.fori_loop(0, (ACC_ROWS - N_NODES) // CHUNK, ztail, 0)

        @pl.when(c == 1)
        def _():
            def zslice(t, carry):
                pltpu.sync_copy(rows_v, acc.at[pl.ds(s * RPT + t * CHUNK, CHUNK)])
                return carry

            lax.fori_loop(0, RPT // CHUNK, zslice, 0)

        plsc.subcore_barrier()

        pltpu.async_copy(h_hbm.at[src_v.at[0]], rows_v, sem0)

        def step(jj, carry):
            j = 2 * jj
            pltpu.async_copy(h_hbm.at[src_v.at[j + 1]], rows_w, sem1)
            pltpu.make_async_copy(h_hbm.at[src_v.at[j]], rows_v, sem0).wait()
            pltpu.sync_copy(rows_v, acc.at[dst_v.at[j]], add=True)

            @pl.when(j + 2 < NCHUNK)
            def _():
                pltpu.async_copy(h_hbm.at[src_v.at[j + 2]], rows_v, sem0)

            pltpu.make_async_copy(h_hbm.at[src_v.at[j + 1]], rows_w, sem1).wait()
            pltpu.sync_copy(rows_w, acc.at[dst_v.at[j + 1]], add=True)
            return carry

        lax.fori_loop(0, NCHUNK // 2, step, 0)
        pltpu.make_async_copy(h_hbm.at[src_v.at[NCHUNK - 1]], rows_v, sem0).wait()
        pltpu.sync_copy(rows_v, acc.at[dst_v.at[NCHUNK - 1]], add=True)
        plsc.subcore_barrier()
        pltpu.sync_copy(
            acc.at[pl.ds(s * RPT, RPT)], out_hbm.at[c, pl.ds(s * RPT, RPT)]
        )

    return agg


_agg128 = _make_agg(128)
_agg64 = _make_agg(64)


def _mm1_body(x_ref, w_ref, dp_ref, o_ref):
    deg = dp_ref[0] + dp_ref[1] + 1.0
    dis = lax.rsqrt(deg)
    h = jnp.dot(x_ref[...], w_ref[...], preferred_element_type=jnp.float32)
    o_ref[...] = h * dis


def _mm1(x, W1, dpr):
    return pl.pallas_call(
        _mm1_body,
        grid=(NBLK,),
        in_specs=[
            pl.BlockSpec((BLK, 128), lambda i: (i, 0)),
            pl.BlockSpec((128, 128), lambda i: (0, 0)),
            pl.BlockSpec((NC, BLK, 1), lambda i: (0, i, 0)),
        ],
        out_specs=pl.BlockSpec((BLK, 128), lambda i: (i, 0)),
        out_shape=jax.ShapeDtypeStruct((ACC_ROWS, 128), jnp.float32),
    )(x, W1, dpr)


def _mm2_body(p_ref, dp_ref, b1_ref, w_ref, o_ref):
    deg = dp_ref[0] + dp_ref[1] + 1.0
    dis = lax.rsqrt(deg)
    aggd = p_ref[0] + p_ref[1]
    h = jnp.maximum(aggd * dis + b1_ref[...], 0.0)
    o_ref[...] = jnp.dot(h, w_ref[...], preferred_element_type=jnp.float32) * dis


def _mm2(p, dpr, b1, W2):
    return pl.pallas_call(
        _mm2_body,
        grid=(NBLK,),
        in_specs=[
            pl.BlockSpec((NC, BLK, 128), lambda i: (0, i, 0)),
            pl.BlockSpec((NC, BLK, 1), lambda i: (0, i, 0)),
            pl.BlockSpec((1, 128), lambda i: (0, 0)),
            pl.BlockSpec((128, 64), lambda i: (0, 0)),
        ],
        out_specs=pl.BlockSpec((BLK, 64), lambda i: (i, 0)),
        out_shape=jax.ShapeDtypeStruct((ACC_ROWS, 64), jnp.float32),
    )(p, dpr, b1, W2)


def _fin_body(q_ref, dp_ref, b2_ref, o_ref):
    deg = dp_ref[0] + dp_ref[1] + 1.0
    dis = lax.rsqrt(deg)
    o_ref[...] = (q_ref[0] + q_ref[1]) * dis + b2_ref[...]


def _fin(q, dpr, b2):
    return pl.pallas_call(
        _fin_body,
        grid=(NBLK,),
        in_specs=[
            pl.BlockSpec((NC, BLK, 64), lambda i: (0, i, 0)),
            pl.BlockSpec((NC, BLK, 1), lambda i: (0, i, 0)),
            pl.BlockSpec((1, 64), lambda i: (0, 0)),
        ],
        out_specs=pl.BlockSpec((BLK, 64), lambda i: (i, 0)),
        out_shape=jax.ShapeDtypeStruct((N_NODES, 64), jnp.float32),
    )(q, dpr, b2)


@jax.jit
def kernel(x, edge_index, W1, b1, W2, b2):
    src = edge_index[0].astype(jnp.int32).reshape(NW, NCHUNK, CHUNK)
    dst = edge_index[1].astype(jnp.int32).reshape(NW, NCHUNK, CHUNK)
    dp = _deg_kernel(dst)
    dpr = dp.reshape(NC, ACC_ROWS, 1)
    h1p = _mm1(x, W1, dpr)
    p = _agg128(h1p, src, dst)
    h2p = _mm2(p, dpr, b1.reshape(1, 128), W2)
    q = _agg64(h2p, src, dst)
    return _fin(q, dpr, b2.reshape(1, 64))


# speedup vs baseline: 33.1896x; 1.1876x over previous
import functools

import jax
import jax.numpy as jnp
from jax import lax
from jax.experimental import pallas as pl
from jax.experimental.pallas import tpu as pltpu
from jax.experimental.pallas import tpu_sc as plsc

N_NODES = 10000
N_EDGES = 320000
NC = 2
NS = 16
NW = NC * NS
EPT = N_EDGES // NW
CHUNK = 80
NCHUNK = EPT // CHUNK
ACC_ROWS = 10240
RPT = ACC_ROWS // NS
N_INIT = N_NODES // NS
BLK = 256
NBLK = ACC_ROWS // BLK

_sc_mesh = functools.partial(
    plsc.VectorSubcoreMesh, core_axis_name="c", subcore_axis_name="s"
)


@functools.partial(
    pl.kernel,
    out_type=jax.ShapeDtypeStruct((NC, ACC_ROWS), jnp.float32),
    mesh=_sc_mesh(),
    scratch_types=[
        pltpu.VMEM((NCHUNK, CHUNK), jnp.int32),
        pltpu.VMEM((CHUNK,), jnp.float32),
        pltpu.VMEM((RPT,), jnp.float32),
        pltpu.VMEM_SHARED((ACC_ROWS,), jnp.float32),
        pltpu.SemaphoreType.DMA,
    ],
)
def _deg_kernel(dst_hbm, out_hbm, dst_v, ones_v, zero_v, acc, sem):
    c = lax.axis_index("c")
    s = lax.axis_index("s")
    wid = s * NC + c

    pltpu.sync_copy(dst_hbm.at[wid], dst_v)
    for i in range(CHUNK // 16):
        ones_v[pl.ds(i * 16, 16)] = jnp.full((16,), 1.0, jnp.float32)

    def zfill(i, carry):
        zero_v[pl.ds(i * 16, 16)] = jnp.zeros((16,), jnp.float32)
        return carry

    lax.fori_loop(0, RPT // 16, zfill, 0)
    pltpu.sync_copy(zero_v, acc.at[pl.ds(s * RPT, RPT)])
    plsc.subcore_barrier()

    def step(j, carry):
        pltpu.sync_copy(ones_v, acc.at[dst_v.at[j]], add=True)
        return carry

    lax.fori_loop(0, NCHUNK, step, 0)
    plsc.subcore_barrier()
    pltpu.sync_copy(acc.at[pl.ds(s * RPT, RPT)], out_hbm.at[c, pl.ds(s * RPT, RPT)])


def _make_agg(d, chunk, nbuf):
    nchunk = EPT // chunk
    assert nchunk % nbuf == 0

    @functools.partial(
        pl.kernel,
        out_type=jax.ShapeDtypeStruct((NC, ACC_ROWS, d), jnp.float32),
        mesh=_sc_mesh(),
        scratch_types=[
            pltpu.VMEM((nchunk, chunk), jnp.int32),
            pltpu.VMEM((nchunk, chunk), jnp.int32),
            [pltpu.VMEM((chunk, d), jnp.float32) for _ in range(nbuf)],
            pltpu.VMEM_SHARED((ACC_ROWS, d), jnp.float32),
            pltpu.SemaphoreType.DMA,
        ],
        compiler_params=pltpu.CompilerParams(use_tc_tiling_on_sc=False),
    )
    def agg(h_hbm, src_hbm, dst_hbm, out_hbm, src_v, dst_v, bufs, acc, sem):
        c = lax.axis_index("c")
        s = lax.axis_index("s")
        wid = s * NC + c

        pltpu.sync_copy(src_hbm.at[wid], src_v)
        pltpu.sync_copy(dst_hbm.at[wid], dst_v)

        zbuf = bufs[0]

        def zrow(i, carry):
            for k in range(d // 16):
                zbuf[i, pl.ds(k * 16, 16)] = jnp.zeros((16,), jnp.float32)
            return carry

        lax.fori_loop(0, chunk, zrow, 0)

        @pl.when(c == 0)
        def _():
            @pl.when(s < NS - 1)
            def _():
                pltpu.sync_copy(
                    h_hbm.at[pl.ds(s * RPT, RPT)], acc.at[pl.ds(s * RPT, RPT)]
                )

            @pl.when(s == NS - 1)
            def _():
                last = N_NODES - (NS - 1) * RPT
                pltpu.sync_copy(
                    h_hbm.at[pl.ds((NS - 1) * RPT, last)],
                    acc.at[pl.ds((NS - 1) * RPT, last)],
                )

                def ztail(t, carry):
                    pltpu.sync_copy(
                        zbuf, acc.at[pl.ds(N_NODES + t * chunk, chunk)]
                    )
                    return carry

                lax.fori_loop(0, (ACC_ROWS - N_NODES) // chunk, ztail, 0)

        @pl.when(c == 1)
        def _():
            def zslice(t, carry):
                pltpu.sync_copy(zbuf, acc.at[pl.ds(s * RPT + t * chunk, chunk)])
                return carry

            lax.fori_loop(0, RPT // chunk, zslice, 0)

        for b in range(nbuf):
            pltpu.async_copy(h_hbm.at[src_v.at[b]], bufs[b], sem)
        plsc.subcore_barrier()

        def step(jj, carry):
            for b in range(nbuf):
                j = nbuf * jj + b
                pltpu.make_async_copy(h_hbm.at[src_v.at[j]], bufs[b], sem).wait()
                pltpu.sync_copy(bufs[b], acc.at[dst_v.at[j]], add=True)

                def refire(b=b, j=j):
                    pltpu.async_copy(
                        h_hbm.at[src_v.at[j + nbuf]], bufs[b], sem
                    )

                pl.when(j + nbuf < nchunk)(refire)
            return carry

        lax.fori_loop(0, nchunk // nbuf, step, 0)
        plsc.subcore_barrier()
        pltpu.sync_copy(
            acc.at[pl.ds(s * RPT, RPT)], out_hbm.at[c, pl.ds(s * RPT, RPT)]
        )

    return agg


_CHUNK1, _NBUF1 = 40, 5
_CHUNK2, _NBUF2 = 80, 5
_agg128 = _make_agg(128, _CHUNK1, _NBUF1)
_agg64 = _make_agg(64, _CHUNK2, _NBUF2)


def _mm1_body(x_ref, w_ref, dp_ref, o_ref):
    deg = dp_ref[0] + dp_ref[1] + 1.0
    dis = lax.rsqrt(deg)
    h = jnp.dot(x_ref[...], w_ref[...], preferred_element_type=jnp.float32)
    o_ref[...] = h * dis


def _mm1(x, W1, dpr):
    return pl.pallas_call(
        _mm1_body,
        grid=(NBLK,),
        in_specs=[
            pl.BlockSpec((BLK, 128), lambda i: (i, 0)),
            pl.BlockSpec((128, 128), lambda i: (0, 0)),
            pl.BlockSpec((NC, BLK, 1), lambda i: (0, i, 0)),
        ],
        out_specs=pl.BlockSpec((BLK, 128), lambda i: (i, 0)),
        out_shape=jax.ShapeDtypeStruct((ACC_ROWS, 128), jnp.float32),
    )(x, W1, dpr)


def _mm2_body(p_ref, dp_ref, b1_ref, w_ref, o_ref):
    deg = dp_ref[0] + dp_ref[1] + 1.0
    dis = lax.rsqrt(deg)
    aggd = p_ref[0] + p_ref[1]
    h = jnp.maximum(aggd * dis + b1_ref[...], 0.0)
    o_ref[...] = jnp.dot(h, w_ref[...], preferred_element_type=jnp.float32) * dis


def _mm2(p, dpr, b1, W2):
    return pl.pallas_call(
        _mm2_body,
        grid=(NBLK,),
        in_specs=[
            pl.BlockSpec((NC, BLK, 128), lambda i: (0, i, 0)),
            pl.BlockSpec((NC, BLK, 1), lambda i: (0, i, 0)),
            pl.BlockSpec((1, 128), lambda i: (0, 0)),
            pl.BlockSpec((128, 64), lambda i: (0, 0)),
        ],
        out_specs=pl.BlockSpec((BLK, 64), lambda i: (i, 0)),
        out_shape=jax.ShapeDtypeStruct((ACC_ROWS, 64), jnp.float32),
    )(p, dpr, b1, W2)


def _fin_body(q_ref, dp_ref, b2_ref, o_ref):
    deg = dp_ref[0] + dp_ref[1] + 1.0
    dis = lax.rsqrt(deg)
    o_ref[...] = (q_ref[0] + q_ref[1]) * dis + b2_ref[...]


def _fin(q, dpr, b2):
    return pl.pallas_call(
        _fin_body,
        grid=(NBLK,),
        in_specs=[
            pl.BlockSpec((NC, BLK, 64), lambda i: (0, i, 0)),
            pl.BlockSpec((NC, BLK, 1), lambda i: (0, i, 0)),
            pl.BlockSpec((1, 64), lambda i: (0, 0)),
        ],
        out_specs=pl.BlockSpec((BLK, 64), lambda i: (i, 0)),
        out_shape=jax.ShapeDtypeStruct((N_NODES, 64), jnp.float32),
    )(q, dpr, b2)


@jax.jit
def kernel(x, edge_index, W1, b1, W2, b2):
    src = edge_index[0].astype(jnp.int32)
    dst = edge_index[1].astype(jnp.int32)
    src1 = src.reshape(NW, EPT // _CHUNK1, _CHUNK1)
    dst1 = dst.reshape(NW, EPT // _CHUNK1, _CHUNK1)
    src2 = src.reshape(NW, EPT // _CHUNK2, _CHUNK2)
    dst2 = dst.reshape(NW, EPT // _CHUNK2, _CHUNK2)
    dp = _deg_kernel(dst.reshape(NW, NCHUNK, CHUNK))
    dpr = dp.reshape(NC, ACC_ROWS, 1)
    h1p = _mm1(x, W1, dpr)
    p = _agg128(h1p, src1, dst1)
    h2p = _mm2(p, dpr, b1.reshape(1, 128), W2)
    q = _agg64(h2p, src2, dst2)
    return _fin(q, dpr, b2.reshape(1, 64))


# speedup vs baseline: 34.0331x; 1.0254x over previous
import functools

import jax
import jax.numpy as jnp
from jax import lax
from jax.experimental import pallas as pl
from jax.experimental.pallas import tpu as pltpu
from jax.experimental.pallas import tpu_sc as plsc

N_NODES = 10000
N_EDGES = 320000
NC = 2
NS = 16
NW = NC * NS
EPT = N_EDGES // NW
CHUNK = 80
NCHUNK = EPT // CHUNK
ACC_ROWS = 10240
RPT = ACC_ROWS // NS
N_INIT = N_NODES // NS
BLK = 256
NBLK = ACC_ROWS // BLK

_sc_mesh = functools.partial(
    plsc.VectorSubcoreMesh, core_axis_name="c", subcore_axis_name="s"
)


@functools.partial(
    pl.kernel,
    out_type=jax.ShapeDtypeStruct((NC, ACC_ROWS), jnp.float32),
    mesh=_sc_mesh(),
    scratch_types=[
        pltpu.VMEM((NCHUNK, CHUNK), jnp.int32),
        pltpu.VMEM((CHUNK,), jnp.float32),
        pltpu.VMEM((RPT,), jnp.float32),
        pltpu.VMEM_SHARED((ACC_ROWS,), jnp.float32),
        pltpu.SemaphoreType.DMA,
    ],
)
def _deg_kernel(ei_hbm, out_hbm, dst_v, ones_v, zero_v, acc, sem):
    c = lax.axis_index("c")
    s = lax.axis_index("s")
    wid = s * NC + c

    pltpu.sync_copy(ei_hbm.at[1, wid], dst_v)
    for i in range(CHUNK // 16):
        ones_v[pl.ds(i * 16, 16)] = jnp.full((16,), 1.0, jnp.float32)

    def zfill(i, carry):
        zero_v[pl.ds(i * 16, 16)] = jnp.zeros((16,), jnp.float32)
        return carry

    lax.fori_loop(0, RPT // 16, zfill, 0)
    pltpu.sync_copy(zero_v, acc.at[pl.ds(s * RPT, RPT)])
    plsc.subcore_barrier()

    def step(j, carry):
        pltpu.sync_copy(ones_v, acc.at[dst_v.at[j]], add=True)
        return carry

    lax.fori_loop(0, NCHUNK, step, 0)
    plsc.subcore_barrier()
    pltpu.sync_copy(acc.at[pl.ds(s * RPT, RPT)], out_hbm.at[c, pl.ds(s * RPT, RPT)])


def _make_agg(d, chunk, nbuf):
    nchunk = EPT // chunk
    assert nchunk % nbuf == 0

    @functools.partial(
        pl.kernel,
        out_type=jax.ShapeDtypeStruct((NC, ACC_ROWS, d), jnp.float32),
        mesh=_sc_mesh(),
        scratch_types=[
            pltpu.VMEM((nchunk, chunk), jnp.int32),
            pltpu.VMEM((nchunk, chunk), jnp.int32),
            [pltpu.VMEM((chunk, d), jnp.float32) for _ in range(nbuf)],
            pltpu.VMEM_SHARED((ACC_ROWS, d), jnp.float32),
            pltpu.SemaphoreType.DMA,
        ],
        compiler_params=pltpu.CompilerParams(use_tc_tiling_on_sc=False),
    )
    def agg(h_hbm, ei_hbm, out_hbm, src_v, dst_v, bufs, acc, sem):
        c = lax.axis_index("c")
        s = lax.axis_index("s")
        wid = s * NC + c

        pltpu.sync_copy(ei_hbm.at[0, wid], src_v)
        pltpu.sync_copy(ei_hbm.at[1, wid], dst_v)

        zbuf = bufs[0]

        def zrow(i, carry):
            for k in range(d // 16):
                zbuf[i, pl.ds(k * 16, 16)] = jnp.zeros((16,), jnp.float32)
            return carry

        lax.fori_loop(0, chunk, zrow, 0)

        @pl.when(c == 0)
        def _():
            @pl.when(s < NS - 1)
            def _():
                pltpu.sync_copy(
                    h_hbm.at[pl.ds(s * RPT, RPT)], acc.at[pl.ds(s * RPT, RPT)]
                )

            @pl.when(s == NS - 1)
            def _():
                last = N_NODES - (NS - 1) * RPT
                pltpu.sync_copy(
                    h_hbm.at[pl.ds((NS - 1) * RPT, last)],
                    acc.at[pl.ds((NS - 1) * RPT, last)],
                )

                def ztail(t, carry):
                    pltpu.sync_copy(
                        zbuf, acc.at[pl.ds(N_NODES + t * chunk, chunk)]
                    )
                    return carry

                lax.fori_loop(0, (ACC_ROWS - N_NODES) // chunk, ztail, 0)

        @pl.when(c == 1)
        def _():
            def zslice(t, carry):
                pltpu.sync_copy(zbuf, acc.at[pl.ds(s * RPT + t * chunk, chunk)])
                return carry

            lax.fori_loop(0, RPT // chunk, zslice, 0)

        for b in range(nbuf):
            pltpu.async_copy(h_hbm.at[src_v.at[b]], bufs[b], sem)
        plsc.subcore_barrier()

        def step(jj, carry):
            for b in range(nbuf):
                j = nbuf * jj + b
                pltpu.make_async_copy(h_hbm.at[src_v.at[j]], bufs[b], sem).wait()
                pltpu.sync_copy(bufs[b], acc.at[dst_v.at[j]], add=True)

                def refire(b=b, j=j):
                    pltpu.async_copy(
                        h_hbm.at[src_v.at[j + nbuf]], bufs[b], sem
                    )

                pl.when(j + nbuf < nchunk)(refire)
            return carry

        lax.fori_loop(0, nchunk // nbuf, step, 0)
        plsc.subcore_barrier()
        pltpu.sync_copy(
            acc.at[pl.ds(s * RPT, RPT)], out_hbm.at[c, pl.ds(s * RPT, RPT)]
        )

    return agg


_CHUNK1, _NBUF1 = 40, 5
_CHUNK2, _NBUF2 = 40, 10
_agg128 = _make_agg(128, _CHUNK1, _NBUF1)
_agg64 = _make_agg(64, _CHUNK2, _NBUF2)


def _mm1_body(x_ref, w_ref, dp_ref, o_ref):
    deg = dp_ref[0] + dp_ref[1] + 1.0
    dis = lax.rsqrt(deg)
    h = jnp.dot(x_ref[...], w_ref[...], preferred_element_type=jnp.float32)
    o_ref[...] = h * dis


def _mm1(x, W1, dpr):
    return pl.pallas_call(
        _mm1_body,
        grid=(NBLK,),
        in_specs=[
            pl.BlockSpec((BLK, 128), lambda i: (i, 0)),
            pl.BlockSpec((128, 128), lambda i: (0, 0)),
            pl.BlockSpec((NC, BLK, 1), lambda i: (0, i, 0)),
        ],
        out_specs=pl.BlockSpec((BLK, 128), lambda i: (i, 0)),
        out_shape=jax.ShapeDtypeStruct((ACC_ROWS, 128), jnp.float32),
    )(x, W1, dpr)


def _mm2_body(p_ref, dp_ref, b1_ref, w_ref, o_ref):
    deg = dp_ref[0] + dp_ref[1] + 1.0
    dis = lax.rsqrt(deg)
    aggd = p_ref[0] + p_ref[1]
    h = jnp.maximum(aggd * dis + b1_ref[...], 0.0)
    o_ref[...] = jnp.dot(h, w_ref[...], preferred_element_type=jnp.float32) * dis


def _mm2(p, dpr, b1, W2):
    return pl.pallas_call(
        _mm2_body,
        grid=(NBLK,),
        in_specs=[
            pl.BlockSpec((NC, BLK, 128), lambda i: (0, i, 0)),
            pl.BlockSpec((NC, BLK, 1), lambda i: (0, i, 0)),
            pl.BlockSpec((1, 128), lambda i: (0, 0)),
            pl.BlockSpec((128, 64), lambda i: (0, 0)),
        ],
        out_specs=pl.BlockSpec((BLK, 64), lambda i: (i, 0)),
        out_shape=jax.ShapeDtypeStruct((ACC_ROWS, 64), jnp.float32),
    )(p, dpr, b1, W2)


def _fin_body(q_ref, dp_ref, b2_ref, o_ref):
    deg = dp_ref[0] + dp_ref[1] + 1.0
    dis = lax.rsqrt(deg)
    o_ref[...] = (q_ref[0] + q_ref[1]) * dis + b2_ref[...]


def _fin(q, dpr, b2):
    return pl.pallas_call(
        _fin_body,
        grid=(NBLK,),
        in_specs=[
            pl.BlockSpec((NC, BLK, 64), lambda i: (0, i, 0)),
            pl.BlockSpec((NC, BLK, 1), lambda i: (0, i, 0)),
            pl.BlockSpec((1, 64), lambda i: (0, 0)),
        ],
        out_specs=pl.BlockSpec((BLK, 64), lambda i: (i, 0)),
        out_shape=jax.ShapeDtypeStruct((N_NODES, 64), jnp.float32),
    )(q, dpr, b2)


@jax.jit
def kernel(x, edge_index, W1, b1, W2, b2):
    ei = edge_index.astype(jnp.int32)
    ei1 = ei.reshape(2, NW, EPT // _CHUNK1, _CHUNK1)
    ei2 = ei.reshape(2, NW, EPT // _CHUNK2, _CHUNK2)
    dp = _deg_kernel(ei.reshape(2, NW, NCHUNK, CHUNK))
    dpr = dp.reshape(NC, ACC_ROWS, 1)
    h1p = _mm1(x, W1, dpr)
    p = _agg128(h1p, ei1)
    h2p = _mm2(p, dpr, b1.reshape(1, 128), W2)
    q = _agg64(h2p, ei2)
    return _fin(q, dpr, b2.reshape(1, 64))


# speedup vs baseline: 38.6964x; 1.1370x over previous
import functools

import jax
import jax.numpy as jnp
from jax import lax
from jax.experimental import pallas as pl
from jax.experimental.pallas import tpu as pltpu
from jax.experimental.pallas import tpu_sc as plsc

N_NODES = 10000
N_EDGES = 320000
NC = 2
NS = 16
NW = NC * NS
EPT = N_EDGES // NW
CHUNK = 40
NCHUNK = EPT // CHUNK
ACC_ROWS = 10240
RPT = ACC_ROWS // NS
N_INIT = N_NODES // NS
BLK = 1024
NBLK = ACC_ROWS // BLK

_sc_mesh = functools.partial(
    plsc.VectorSubcoreMesh, core_axis_name="c", subcore_axis_name="s"
)


@functools.partial(
    pl.kernel,
    out_type=jax.ShapeDtypeStruct((NC, ACC_ROWS), jnp.float32),
    mesh=_sc_mesh(),
    scratch_types=[
        pltpu.VMEM((NCHUNK, CHUNK), jnp.int32),
        pltpu.VMEM((80,), jnp.float32),
        pltpu.VMEM((RPT,), jnp.float32),
        pltpu.VMEM_SHARED((ACC_ROWS,), jnp.float32),
        pltpu.SemaphoreType.DMA,
    ],
)
def _deg_kernel(ei_hbm, out_hbm, dst_v, ones_v, zero_v, acc, sem):
    c = lax.axis_index("c")
    s = lax.axis_index("s")
    wid = s * NC + c

    pltpu.sync_copy(ei_hbm.at[1, wid], dst_v)
    for i in range(5):
        ones_v[pl.ds(i * 16, 16)] = jnp.full((16,), 1.0, jnp.float32)

    def zfill(i, carry):
        zero_v[pl.ds(i * 16, 16)] = jnp.zeros((16,), jnp.float32)
        return carry

    lax.fori_loop(0, RPT // 16, zfill, 0)
    pltpu.sync_copy(zero_v, acc.at[pl.ds(s * RPT, RPT)])
    plsc.subcore_barrier()

    def step(j, carry):
        pltpu.sync_copy(ones_v.at[pl.ds(0, CHUNK)], acc.at[dst_v.at[j]], add=True)
        return carry

    lax.fori_loop(0, NCHUNK, step, 0)
    plsc.subcore_barrier()
    pltpu.sync_copy(acc.at[pl.ds(s * RPT, RPT)], out_hbm.at[c, pl.ds(s * RPT, RPT)])


def _make_agg(d, chunk, nbuf):
    nchunk = EPT // chunk
    assert nchunk % nbuf == 0

    @functools.partial(
        pl.kernel,
        out_type=jax.ShapeDtypeStruct((NC, ACC_ROWS, d), jnp.float32),
        mesh=_sc_mesh(),
        scratch_types=[
            pltpu.VMEM((nchunk, chunk), jnp.int32),
            pltpu.VMEM((nchunk, chunk), jnp.int32),
            [pltpu.VMEM((chunk, d), jnp.float32) for _ in range(nbuf)],
            pltpu.VMEM_SHARED((ACC_ROWS, d), jnp.float32),
            pltpu.SemaphoreType.DMA,
        ],
        compiler_params=pltpu.CompilerParams(use_tc_tiling_on_sc=False),
    )
    def agg(h_hbm, ei_hbm, out_hbm, src_v, dst_v, bufs, acc, sem):
        c = lax.axis_index("c")
        s = lax.axis_index("s")
        wid = s * NC + c

        pltpu.sync_copy(ei_hbm.at[0, wid], src_v)
        pltpu.sync_copy(ei_hbm.at[1, wid], dst_v)

        zbuf = bufs[0]

        def zrow(i, carry):
            for k in range(d // 16):
                zbuf[i, pl.ds(k * 16, 16)] = jnp.zeros((16,), jnp.float32)
            return carry

        lax.fori_loop(0, chunk, zrow, 0)

        @pl.when(c == 0)
        def _():
            @pl.when(s < NS - 1)
            def _():
                pltpu.sync_copy(
                    h_hbm.at[pl.ds(s * RPT, RPT)], acc.at[pl.ds(s * RPT, RPT)]
                )

            @pl.when(s == NS - 1)
            def _():
                last = N_NODES - (NS - 1) * RPT
                pltpu.sync_copy(
                    h_hbm.at[pl.ds((NS - 1) * RPT, last)],
                    acc.at[pl.ds((NS - 1) * RPT, last)],
                )

                def ztail(t, carry):
                    pltpu.sync_copy(
                        zbuf, acc.at[pl.ds(N_NODES + t * chunk, chunk)]
                    )
                    return carry

                lax.fori_loop(0, (ACC_ROWS - N_NODES) // chunk, ztail, 0)

        @pl.when(c == 1)
        def _():
            def zslice(t, carry):
                pltpu.sync_copy(zbuf, acc.at[pl.ds(s * RPT + t * chunk, chunk)])
                return carry

            lax.fori_loop(0, RPT // chunk, zslice, 0)

        for b in range(nbuf):
            pltpu.async_copy(h_hbm.at[src_v.at[b]], bufs[b], sem)
        plsc.subcore_barrier()

        def step(jj, carry):
            for b in range(nbuf):
                j = nbuf * jj + b
                pltpu.make_async_copy(h_hbm.at[src_v.at[j]], bufs[b], sem).wait()
                pltpu.sync_copy(bufs[b], acc.at[dst_v.at[j]], add=True)

                def refire(b=b, j=j):
                    pltpu.async_copy(
                        h_hbm.at[src_v.at[j + nbuf]], bufs[b], sem
                    )

                pl.when(j + nbuf < nchunk)(refire)
            return carry

        lax.fori_loop(0, nchunk // nbuf, step, 0)
        plsc.subcore_barrier()
        pltpu.sync_copy(
            acc.at[pl.ds(s * RPT, RPT)], out_hbm.at[c, pl.ds(s * RPT, RPT)]
        )

    return agg


_CHUNK1, _NBUF1 = 40, 5
_CHUNK2, _NBUF2 = 40, 10
_agg128 = _make_agg(128, _CHUNK1, _NBUF1)
_agg64 = _make_agg(64, _CHUNK2, _NBUF2)


def _mm1_body(x_ref, w_ref, dp_ref, o_ref):
    deg = dp_ref[0] + dp_ref[1] + 1.0
    dis = lax.rsqrt(deg)
    h = jnp.dot(x_ref[...], w_ref[...], preferred_element_type=jnp.float32)
    o_ref[...] = h * dis


def _mm1(x, W1, dpr):
    return pl.pallas_call(
        _mm1_body,
        grid=(NBLK,),
        in_specs=[
            pl.BlockSpec((BLK, 128), lambda i: (i, 0)),
            pl.BlockSpec((128, 128), lambda i: (0, 0)),
            pl.BlockSpec((NC, BLK, 1), lambda i: (0, i, 0)),
        ],
        out_specs=pl.BlockSpec((BLK, 128), lambda i: (i, 0)),
        out_shape=jax.ShapeDtypeStruct((ACC_ROWS, 128), jnp.float32),
    )(x, W1, dpr)


def _mm2_body(p_ref, dp_ref, b1_ref, w_ref, o_ref):
    deg = dp_ref[0] + dp_ref[1] + 1.0
    dis = lax.rsqrt(deg)
    aggd = p_ref[0] + p_ref[1]
    h = jnp.maximum(aggd * dis + b1_ref[...], 0.0)
    o_ref[...] = jnp.dot(h, w_ref[...], preferred_element_type=jnp.float32) * dis


def _mm2(p, dpr, b1, W2):
    return pl.pallas_call(
        _mm2_body,
        grid=(NBLK,),
        in_specs=[
            pl.BlockSpec((NC, BLK, 128), lambda i: (0, i, 0)),
            pl.BlockSpec((NC, BLK, 1), lambda i: (0, i, 0)),
            pl.BlockSpec((1, 128), lambda i: (0, 0)),
            pl.BlockSpec((128, 64), lambda i: (0, 0)),
        ],
        out_specs=pl.BlockSpec((BLK, 64), lambda i: (i, 0)),
        out_shape=jax.ShapeDtypeStruct((ACC_ROWS, 64), jnp.float32),
    )(p, dpr, b1, W2)


def _fin_body(q_ref, dp_ref, b2_ref, o_ref):
    deg = dp_ref[0] + dp_ref[1] + 1.0
    dis = lax.rsqrt(deg)
    o_ref[...] = (q_ref[0] + q_ref[1]) * dis + b2_ref[...]


def _fin(q, dpr, b2):
    return pl.pallas_call(
        _fin_body,
        grid=(NBLK,),
        in_specs=[
            pl.BlockSpec((NC, BLK, 64), lambda i: (0, i, 0)),
            pl.BlockSpec((NC, BLK, 1), lambda i: (0, i, 0)),
            pl.BlockSpec((1, 64), lambda i: (0, 0)),
        ],
        out_specs=pl.BlockSpec((BLK, 64), lambda i: (i, 0)),
        out_shape=jax.ShapeDtypeStruct((N_NODES, 64), jnp.float32),
    )(q, dpr, b2)


@jax.jit
def kernel(x, edge_index, W1, b1, W2, b2):
    ei = edge_index.astype(jnp.int32)
    ei1 = ei.reshape(2, NW, EPT // _CHUNK1, _CHUNK1)
    ei2 = ei1
    dp = _deg_kernel(ei1)
    dpr = dp.reshape(NC, ACC_ROWS, 1)
    h1p = _mm1(x, W1, dpr)
    p = _agg128(h1p, ei1)
    h2p = _mm2(p, dpr, b1.reshape(1, 128), W2)
    q = _agg64(h2p, ei2)
    return _fin(q, dpr, b2.reshape(1, 64))
